# Initial kernel scaffold; baseline (speedup 1.0000x reference)
import jax
import jax.numpy as jnp
from jax.experimental import pallas as pl


def kernel(z, pos, edge_index, batch, emb_table, m1_w, m1_b, m2_w, m2_b, u1_w, u1_b, u2_w, u2_b, vmix_w, final_w):
    raise NotImplementedError("write your pallas kernel here")


# speedup vs baseline: 12.5145x; 12.5145x over previous
import functools

import jax
import jax.numpy as jnp
from jax import lax
from jax.experimental import pallas as pl
from jax.experimental.pallas import tpu as pltpu
from jax.experimental.pallas import tpu_sc as plsc

H = 128
N_NODES = 10000
N_EDGES = 160000
NUM_MOLS = 64
EPSILON = 1e-8

NC = 2
NS = 16
NW = NC * NS

N_PAD = 10240
E_PAD = 163840
EC = 128
G_CH = 40
S_CH = 80
NZ_CH = 8
NZ_W = 40

BE = 2048
BN = 1024

_mesh = plsc.VectorSubcoreMesh(
    core_axis_name="c", subcore_axis_name="s", num_cores=NC, num_subcores=NS)

f32 = jnp.float32
i32 = jnp.int32


def _wid():
    return lax.axis_index("s") * NC + lax.axis_index("c")


@functools.partial(
    pl.kernel,
    out_type=[
        jax.ShapeDtypeStruct((N_PAD, H), f32),
        jax.ShapeDtypeStruct((E_PAD, H), f32),
        jax.ShapeDtypeStruct((E_PAD, H), f32),
    ],
    mesh=_mesh,
    scratch_types=[
        pltpu.VMEM((NZ_CH, NZ_W), i32),
        pltpu.VMEM((NZ_W, H), f32),
        pltpu.VMEM((G_CH, EC), i32),
        pltpu.VMEM((G_CH, EC), i32),
        pltpu.VMEM((EC, H), f32),
        pltpu.VMEM((EC, H), f32),
        pltpu.SemaphoreType.DMA,
        pltpu.SemaphoreType.DMA,
    ],
)
def _sc_gather_init(emb_hbm, z2_hbm, pos_hbm, row2_hbm, col2_hbm,
                    x0_hbm, posr_hbm, posc_hbm,
                    zidx, zrows, ridx, cidx, rbuf, cbuf, sem_r, sem_c):
    w = _wid()
    pltpu.sync_copy(z2_hbm.at[pl.ds(w * NZ_CH, NZ_CH)], zidx)

    def zbody(j, carry):
        pltpu.async_copy(emb_hbm.at[zidx.at[j]], zrows, sem_r).wait()
        pltpu.sync_copy(
            zrows, x0_hbm.at[pl.ds(w * (NZ_CH * NZ_W) + j * NZ_W, NZ_W)])
        return carry

    lax.fori_loop(0, NZ_CH, zbody, 0)

    pltpu.sync_copy(row2_hbm.at[pl.ds(w * G_CH, G_CH)], ridx)
    pltpu.sync_copy(col2_hbm.at[pl.ds(w * G_CH, G_CH)], cidx)

    def ebody(j, carry):
        cr = pltpu.async_copy(pos_hbm.at[ridx.at[j]], rbuf, sem_r)
        cc = pltpu.async_copy(pos_hbm.at[cidx.at[j]], cbuf, sem_c)
        cr.wait()
        cc.wait()
        base = w * (G_CH * EC) + j * EC
        pltpu.sync_copy(rbuf, posr_hbm.at[pl.ds(base, EC)])
        pltpu.sync_copy(cbuf, posc_hbm.at[pl.ds(base, EC)])
        return carry

    lax.fori_loop(0, G_CH, ebody, 0)


@functools.partial(
    pl.kernel,
    out_type=[
        jax.ShapeDtypeStruct((E_PAD, H), f32),
        jax.ShapeDtypeStruct((E_PAD, H), f32),
    ],
    mesh=_mesh,
    scratch_types=[
        pltpu.VMEM((G_CH, EC), i32),
        pltpu.VMEM((G_CH, EC), i32),
        pltpu.VMEM((EC, H), f32),
        pltpu.VMEM((EC, H), f32),
        pltpu.SemaphoreType.DMA,
        pltpu.SemaphoreType.DMA,
    ],
)
def _sc_gather_x(x_hbm, row2_hbm, col2_hbm, xr_hbm, xc_hbm,
                 ridx, cidx, rbuf, cbuf, sem_r, sem_c):
    w = _wid()
    pltpu.sync_copy(row2_hbm.at[pl.ds(w * G_CH, G_CH)], ridx)
    pltpu.sync_copy(col2_hbm.at[pl.ds(w * G_CH, G_CH)], cidx)

    def body(j, carry):
        cr = pltpu.async_copy(x_hbm.at[ridx.at[j]], rbuf, sem_r)
        cc = pltpu.async_copy(x_hbm.at[cidx.at[j]], cbuf, sem_c)
        cr.wait()
        cc.wait()
        base = w * (G_CH * EC) + j * EC
        pltpu.sync_copy(rbuf, xr_hbm.at[pl.ds(base, EC)])
        pltpu.sync_copy(cbuf, xc_hbm.at[pl.ds(base, EC)])
        return carry

    lax.fori_loop(0, G_CH, body, 0)


@functools.partial(
    pl.kernel,
    out_type=jax.ShapeDtypeStruct((4, N_PAD, H), f32),
    mesh=_mesh,
    scratch_types=[
        pltpu.VMEM_SHARED((N_PAD, H), f32),
        pltpu.VMEM((S_CH, EC), i32),
        pltpu.VMEM((EC, H), f32),
    ],
)
def _sc_scatter(msgs3_hbm, row2_hbm, zeros_hbm, agg3_hbm, spmem, idx_v, mbuf):
    core = lax.axis_index("c")
    sid = lax.axis_index("s")
    pltpu.sync_copy(row2_hbm.at[pl.ds(sid * S_CH, S_CH)], idx_v)
    rows0 = sid * (N_PAD // NS)
    nrows = N_PAD // NS
    for fc in range(2):
        ch = core * 2 + fc
        pltpu.sync_copy(zeros_hbm.at[pl.ds(rows0, nrows)],
                        spmem.at[pl.ds(rows0, nrows)])
        plsc.subcore_barrier()

        def body(j, carry):
            pltpu.sync_copy(
                msgs3_hbm.at[ch, pl.ds(sid * (S_CH * EC) + j * EC, EC)], mbuf)
            pltpu.sync_copy(mbuf, spmem.at[idx_v.at[j]], add=True)
            return carry

        lax.fori_loop(0, S_CH, body, 0)
        plsc.subcore_barrier()
        pltpu.sync_copy(spmem.at[pl.ds(rows0, nrows)],
                        agg3_hbm.at[ch, pl.ds(rows0, nrows)])
        plsc.subcore_barrier()


def _geom_body(pr_ref, pc_ref, out_ref):
    r = pc_ref[...] - pr_ref[...]
    dist = jnp.sqrt(jnp.sum(r * r, axis=1, keepdims=True))
    inv = 1.0 / jnp.maximum(dist, EPSILON)
    d = r * inv
    out_ref[...] = jnp.concatenate(
        [dist, d[:, 0:3], jnp.zeros((BE, 12), f32)], axis=1)


def _tc_geom(posr, posc):
    grid = (E_PAD // BE,)
    return pl.pallas_call(
        _geom_body,
        grid=grid,
        in_specs=[
            pl.BlockSpec((BE, H), lambda i: (i, 0)),
            pl.BlockSpec((BE, H), lambda i: (i, 0)),
        ],
        out_specs=pl.BlockSpec((BE, 16), lambda i: (i, 0)),
        out_shape=jax.ShapeDtypeStruct((E_PAD, 16), f32),
    )(posr, posc)


def _edge_body(xr_ref, xc_ref, geom_ref, w1a_ref, w1b_ref, w1c_ref,
               b1_ref, w2_ref, b2_ref, out_ref):
    geom = geom_ref[...]
    dist = geom[:, 0:1]
    d = geom[:, 1:4]
    m1 = jnp.dot(xr_ref[...], w1a_ref[...], preferred_element_type=f32)
    m1 = m1 + jnp.dot(xc_ref[...], w1b_ref[...], preferred_element_type=f32)
    m1 = m1 + dist * w1c_ref[...] + b1_ref[...]
    h1 = m1 * jax.nn.sigmoid(m1)
    m2 = jnp.dot(h1, w2_ref[...], preferred_element_type=f32) + b2_ref[...]
    msg = m2 * jax.nn.sigmoid(m2)
    gate = msg[:, :H]
    out_ref[0] = msg[:, H:]
    out_ref[1] = gate * d[:, 0:1]
    out_ref[2] = gate * d[:, 1:2]
    out_ref[3] = gate * d[:, 2:3]


def _tc_edge(xr, xc, geom, w1a, w1b, w1c, b1, w2, b2):
    grid = (E_PAD // BE,)
    full = lambda shape: pl.BlockSpec(shape, lambda i: tuple(0 for _ in shape))
    return pl.pallas_call(
        _edge_body,
        grid=grid,
        in_specs=[
            pl.BlockSpec((BE, H), lambda i: (i, 0)),
            pl.BlockSpec((BE, H), lambda i: (i, 0)),
            pl.BlockSpec((BE, 16), lambda i: (i, 0)),
            full((H, 2 * H)),
            full((H, 2 * H)),
            full((1, 2 * H)),
            full((1, 2 * H)),
            full((2 * H, 2 * H)),
            full((1, 2 * H)),
        ],
        out_specs=pl.BlockSpec((4, BE, H), lambda i: (0, i, 0)),
        out_shape=jax.ShapeDtypeStruct((4, E_PAD, H), f32),
    )(xr, xc, geom, w1a, w1b, w1c, b1, w2, b2)


def _node_body(x_ref, agg_ref, v_ref, u1a_ref, u1b_ref, b1_ref, u2_ref,
               b2_ref, vmix_ref, xn_ref, vn_ref):
    x = x_ref[...]
    aggs = agg_ref[0]
    h = (jnp.dot(x, u1a_ref[...], preferred_element_type=f32)
         + jnp.dot(aggs, u1b_ref[...], preferred_element_type=f32)
         + b1_ref[...])
    h = h * jax.nn.sigmoid(h)
    xn_ref[...] = x + jnp.dot(h, u2_ref[...], preferred_element_type=f32) + b2_ref[...]
    for c in range(3):
        vc = v_ref[:, c * H:(c + 1) * H]
        vn_ref[:, c * H:(c + 1) * H] = (
            vc + agg_ref[c + 1]
            + jnp.dot(vc, vmix_ref[...], preferred_element_type=f32))


def _tc_node(x, agg3, v, u1a, u1b, b1, u2, b2, vmix):
    grid = (N_PAD // BN,)
    full = lambda shape: pl.BlockSpec(shape, lambda i: tuple(0 for _ in shape))
    return pl.pallas_call(
        _node_body,
        grid=grid,
        in_specs=[
            pl.BlockSpec((BN, H), lambda i: (i, 0)),
            pl.BlockSpec((4, BN, H), lambda i: (0, i, 0)),
            pl.BlockSpec((BN, 3 * H), lambda i: (i, 0)),
            full((H, H)),
            full((H, H)),
            full((1, H)),
            full((H, H)),
            full((1, H)),
            full((H, H)),
        ],
        out_specs=[
            pl.BlockSpec((BN, H), lambda i: (i, 0)),
            pl.BlockSpec((BN, 3 * H), lambda i: (i, 0)),
        ],
        out_shape=[
            jax.ShapeDtypeStruct((N_PAD, H), f32),
            jax.ShapeDtypeStruct((N_PAD, 3 * H), f32),
        ],
    )(x, agg3, v, u1a, u1b, b1, u2, b2, vmix)


def _pool_body(batch_ref, v_ref, fw_ref, e3_ref, out_ref):
    i = pl.program_id(0)

    @pl.when(i == 0)
    def _():
        out_ref[...] = jnp.zeros_like(out_ref)

    b = batch_ref[0]
    mids = lax.broadcasted_iota(i32, (NUM_MOLS, BN), 0)
    mask = (mids == b).astype(f32)
    fw = fw_ref[...]
    ndp = jnp.zeros((BN, H), f32)
    for c in range(3):
        nd = jnp.sum(v_ref[:, c * H:(c + 1) * H] * fw, axis=1, keepdims=True)
        ndp = ndp + nd * e3_ref[c:c + 1, :]
    out_ref[...] += jnp.dot(mask, ndp, preferred_element_type=f32)


def _tc_pool(batch3, v, fw_row, e3):
    grid = (N_PAD // BN,)
    return pl.pallas_call(
        _pool_body,
        grid=grid,
        in_specs=[
            pl.BlockSpec((1, 1, BN), lambda i: (i, 0, 0)),
            pl.BlockSpec((BN, 3 * H), lambda i: (i, 0)),
            pl.BlockSpec((1, H), lambda i: (0, 0)),
            pl.BlockSpec((8, H), lambda i: (0, 0)),
        ],
        out_specs=pl.BlockSpec((NUM_MOLS, H), lambda i: (0, 0)),
        out_shape=jax.ShapeDtypeStruct((NUM_MOLS, H), f32),
    )(batch3, v, fw_row, e3)


def kernel(z, pos, edge_index, batch, emb_table, m1_w, m1_b, m2_w, m2_b,
           u1_w, u1_b, u2_w, u2_b, vmix_w, final_w):
    n_layers = m1_w.shape[0]

    row = edge_index[0].astype(i32)
    col = edge_index[1].astype(i32)
    epad = E_PAD - N_EDGES
    row_p = jnp.concatenate([row, jnp.full((epad,), N_NODES, i32)])
    col_p = jnp.concatenate([col, jnp.full((epad,), N_NODES, i32)])
    row2 = row_p.reshape(E_PAD // EC, EC)
    col2 = col_p.reshape(E_PAD // EC, EC)

    npad = N_PAD - N_NODES
    z2 = jnp.concatenate([z.astype(i32), jnp.zeros((npad,), i32)]).reshape(-1, NZ_W)
    pos_pad = jnp.zeros((N_PAD, H), f32).at[:N_NODES, :3].set(pos.astype(f32))
    batch_p = jnp.concatenate([batch.astype(i32), jnp.full((npad,), NUM_MOLS, i32)])
    batch3 = batch_p.reshape(N_PAD // BN, 1, BN)
    zeros_n = jnp.zeros((N_PAD, H), f32)
    fw_row = final_w.astype(f32).reshape(1, H)
    e3 = jnp.eye(8, H, dtype=f32)

    x, posr, posc = _sc_gather_init(emb_table.astype(f32), z2, pos_pad,
                                    row2, col2)
    geom = _tc_geom(posr, posc)

    v = jnp.broadcast_to(pos_pad[:, :3, None], (N_PAD, 3, H)).reshape(N_PAD, 3 * H)

    for l in range(n_layers):
        w1a = m1_w[l, :H, :]
        w1b = m1_w[l, H:2 * H, :]
        w1c = m1_w[l, 2 * H:2 * H + 1, :]
        b1 = m1_b[l].reshape(1, 2 * H)
        w2 = m2_w[l]
        b2 = m2_b[l].reshape(1, 2 * H)
        u1a = u1_w[l, :H, :]
        u1b = u1_w[l, H:, :]
        ub1 = u1_b[l].reshape(1, H)
        u2 = u2_w[l]
        ub2 = u2_b[l].reshape(1, H)
        vmix = vmix_w[l]

        xr, xc = _sc_gather_x(x, row2, col2)
        msgs3 = _tc_edge(xr, xc, geom, w1a, w1b, w1c, b1, w2, b2)
        agg3 = _sc_scatter(msgs3, row2, zeros_n)
        x, v = _tc_node(x, agg3, v, u1a, u1b, ub1, u2, ub2, vmix)

    out = _tc_pool(batch3, v, fw_row, e3)
    return out[:NUM_MOLS, :3]


# speedup vs baseline: 14.7206x; 1.1763x over previous
import functools

import jax
import jax.numpy as jnp
from jax import lax
from jax.experimental import pallas as pl
from jax.experimental.pallas import tpu as pltpu
from jax.experimental.pallas import tpu_sc as plsc

H = 128
N_NODES = 10000
N_EDGES = 160000
NUM_MOLS = 64
EPSILON = 1e-8

NC = 2
NS = 16
NW = NC * NS

N_PAD = 10240
E_PAD = 163840
EC = 128
G_CH = 40
S_CH = 80
NZ_CH = 8
NZ_W = 40

BE = 2048
BN = 1024

_mesh = plsc.VectorSubcoreMesh(
    core_axis_name="c", subcore_axis_name="s", num_cores=NC, num_subcores=NS)

f32 = jnp.float32
i32 = jnp.int32


def _wid():
    return lax.axis_index("s") * NC + lax.axis_index("c")


def _pipelined_gather2(src_hbm, ridx, cidx, out_r_hbm, out_c_hbm, base0,
                       n_chunks, rb, cb, sem_r, sem_c):
    pltpu.async_copy(src_hbm.at[ridx.at[0]], rb[0], sem_r[0])
    pltpu.async_copy(src_hbm.at[cidx.at[0]], cb[0], sem_c[0])

    def body(jj, carry):
        j0 = 2 * jj
        j1 = j0 + 1
        pltpu.async_copy(src_hbm.at[ridx.at[j1]], rb[1], sem_r[1])
        pltpu.async_copy(src_hbm.at[cidx.at[j1]], cb[1], sem_c[1])
        pltpu.make_async_copy(src_hbm.at[ridx.at[j0]], rb[0], sem_r[0]).wait()
        pltpu.make_async_copy(src_hbm.at[cidx.at[j0]], cb[0], sem_c[0]).wait()
        pltpu.sync_copy(rb[0], out_r_hbm.at[pl.ds(base0 + j0 * EC, EC)])
        pltpu.sync_copy(cb[0], out_c_hbm.at[pl.ds(base0 + j0 * EC, EC)])

        @pl.when(jj < n_chunks // 2 - 1)
        def _():
            pltpu.async_copy(src_hbm.at[ridx.at[j0 + 2]], rb[0], sem_r[0])
            pltpu.async_copy(src_hbm.at[cidx.at[j0 + 2]], cb[0], sem_c[0])

        pltpu.make_async_copy(src_hbm.at[ridx.at[j1]], rb[1], sem_r[1]).wait()
        pltpu.make_async_copy(src_hbm.at[cidx.at[j1]], cb[1], sem_c[1]).wait()
        pltpu.sync_copy(rb[1], out_r_hbm.at[pl.ds(base0 + j1 * EC, EC)])
        pltpu.sync_copy(cb[1], out_c_hbm.at[pl.ds(base0 + j1 * EC, EC)])
        return carry

    lax.fori_loop(0, n_chunks // 2, body, 0)


@functools.partial(
    pl.kernel,
    out_type=[
        jax.ShapeDtypeStruct((N_PAD, H), f32),
        jax.ShapeDtypeStruct((E_PAD, H), f32),
        jax.ShapeDtypeStruct((E_PAD, H), f32),
    ],
    mesh=_mesh,
    scratch_types=[
        pltpu.VMEM((NZ_CH, NZ_W), i32),
        pltpu.VMEM((NZ_W, H), f32),
        pltpu.VMEM((G_CH, EC), i32),
        pltpu.VMEM((G_CH, EC), i32),
        pltpu.VMEM((EC, H), f32),
        pltpu.VMEM((EC, H), f32),
        pltpu.VMEM((EC, H), f32),
        pltpu.VMEM((EC, H), f32),
        pltpu.SemaphoreType.DMA,
        pltpu.SemaphoreType.DMA,
        pltpu.SemaphoreType.DMA,
        pltpu.SemaphoreType.DMA,
    ],
)
def _sc_gather_init(emb_hbm, z2_hbm, pos_hbm, row2_hbm, col2_hbm,
                    x0_hbm, posr_hbm, posc_hbm,
                    zidx, zrows, ridx, cidx, rb0, rb1, cb0, cb1,
                    sem_r0, sem_r1, sem_c0, sem_c1):
    w = _wid()
    pltpu.sync_copy(z2_hbm.at[pl.ds(w * NZ_CH, NZ_CH)], zidx)

    def zbody(j, carry):
        pltpu.async_copy(emb_hbm.at[zidx.at[j]], zrows, sem_r0).wait()
        pltpu.sync_copy(
            zrows, x0_hbm.at[pl.ds(w * (NZ_CH * NZ_W) + j * NZ_W, NZ_W)])
        return carry

    lax.fori_loop(0, NZ_CH, zbody, 0)

    pltpu.sync_copy(row2_hbm.at[pl.ds(w * G_CH, G_CH)], ridx)
    pltpu.sync_copy(col2_hbm.at[pl.ds(w * G_CH, G_CH)], cidx)
    _pipelined_gather2(pos_hbm, ridx, cidx, posr_hbm, posc_hbm,
                       w * (G_CH * EC), G_CH, (rb0, rb1), (cb0, cb1),
                       (sem_r0, sem_r1), (sem_c0, sem_c1))


@functools.partial(
    pl.kernel,
    out_type=[
        jax.ShapeDtypeStruct((E_PAD, H), f32),
        jax.ShapeDtypeStruct((E_PAD, H), f32),
    ],
    mesh=_mesh,
    scratch_types=[
        pltpu.VMEM((G_CH, EC), i32),
        pltpu.VMEM((G_CH, EC), i32),
        pltpu.VMEM((EC, H), f32),
        pltpu.VMEM((EC, H), f32),
        pltpu.VMEM((EC, H), f32),
        pltpu.VMEM((EC, H), f32),
        pltpu.SemaphoreType.DMA,
        pltpu.SemaphoreType.DMA,
        pltpu.SemaphoreType.DMA,
        pltpu.SemaphoreType.DMA,
    ],
)
def _sc_gather_x(x_hbm, row2_hbm, col2_hbm, xr_hbm, xc_hbm,
                 ridx, cidx, rb0, rb1, cb0, cb1,
                 sem_r0, sem_r1, sem_c0, sem_c1):
    w = _wid()
    pltpu.sync_copy(row2_hbm.at[pl.ds(w * G_CH, G_CH)], ridx)
    pltpu.sync_copy(col2_hbm.at[pl.ds(w * G_CH, G_CH)], cidx)
    _pipelined_gather2(x_hbm, ridx, cidx, xr_hbm, xc_hbm,
                       w * (G_CH * EC), G_CH, (rb0, rb1), (cb0, cb1),
                       (sem_r0, sem_r1), (sem_c0, sem_c1))


@functools.partial(
    pl.kernel,
    out_type=jax.ShapeDtypeStruct((4, N_PAD, H), f32),
    mesh=_mesh,
    scratch_types=[
        pltpu.VMEM_SHARED((N_PAD, H), f32),
        pltpu.VMEM((S_CH, EC), i32),
        pltpu.VMEM((EC, H), f32),
        pltpu.VMEM((EC, H), f32),
        pltpu.SemaphoreType.DMA,
        pltpu.SemaphoreType.DMA,
    ],
)
def _sc_scatter(msgs3_hbm, row2_hbm, zeros_hbm, agg3_hbm, spmem, idx_v,
                mb0, mb1, sem_m0, sem_m1):
    core = lax.axis_index("c")
    sid = lax.axis_index("s")
    pltpu.sync_copy(row2_hbm.at[pl.ds(sid * S_CH, S_CH)], idx_v)
    rows0 = sid * (N_PAD // NS)
    nrows = N_PAD // NS
    ebase = sid * (S_CH * EC)
    for fc in range(2):
        ch = core * 2 + fc
        pltpu.sync_copy(zeros_hbm.at[pl.ds(rows0, nrows)],
                        spmem.at[pl.ds(rows0, nrows)])
        plsc.subcore_barrier()

        pltpu.async_copy(msgs3_hbm.at[ch, pl.ds(ebase, EC)], mb0, sem_m0)

        def body(jj, carry):
            j0 = 2 * jj
            j1 = j0 + 1
            pltpu.async_copy(
                msgs3_hbm.at[ch, pl.ds(ebase + j1 * EC, EC)], mb1, sem_m1)
            pltpu.make_async_copy(
                msgs3_hbm.at[ch, pl.ds(ebase + j0 * EC, EC)], mb0, sem_m0).wait()
            pltpu.sync_copy(mb0, spmem.at[idx_v.at[j0]], add=True)

            @pl.when(jj < S_CH // 2 - 1)
            def _():
                pltpu.async_copy(
                    msgs3_hbm.at[ch, pl.ds(ebase + (j0 + 2) * EC, EC)],
                    mb0, sem_m0)

            pltpu.make_async_copy(
                msgs3_hbm.at[ch, pl.ds(ebase + j1 * EC, EC)], mb1, sem_m1).wait()
            pltpu.sync_copy(mb1, spmem.at[idx_v.at[j1]], add=True)
            return carry

        lax.fori_loop(0, S_CH // 2, body, 0)
        plsc.subcore_barrier()
        pltpu.sync_copy(spmem.at[pl.ds(rows0, nrows)],
                        agg3_hbm.at[ch, pl.ds(rows0, nrows)])
        plsc.subcore_barrier()


def _geom_body(pr_ref, pc_ref, out_ref):
    r = pc_ref[...] - pr_ref[...]
    dist = jnp.sqrt(jnp.sum(r * r, axis=1, keepdims=True))
    inv = 1.0 / jnp.maximum(dist, EPSILON)
    d = r * inv
    out_ref[...] = jnp.concatenate(
        [dist, d[:, 0:3], jnp.zeros((BE, 12), f32)], axis=1)


def _tc_geom(posr, posc):
    grid = (E_PAD // BE,)
    return pl.pallas_call(
        _geom_body,
        grid=grid,
        in_specs=[
            pl.BlockSpec((BE, H), lambda i: (i, 0)),
            pl.BlockSpec((BE, H), lambda i: (i, 0)),
        ],
        out_specs=pl.BlockSpec((BE, 16), lambda i: (i, 0)),
        out_shape=jax.ShapeDtypeStruct((E_PAD, 16), f32),
    )(posr, posc)


def _edge_body(xr_ref, xc_ref, geom_ref, w1a_ref, w1b_ref, w1c_ref,
               b1_ref, w2_ref, b2_ref, out_ref):
    geom = geom_ref[...]
    dist = geom[:, 0:1]
    d = geom[:, 1:4]
    m1 = jnp.dot(xr_ref[...], w1a_ref[...], preferred_element_type=f32)
    m1 = m1 + jnp.dot(xc_ref[...], w1b_ref[...], preferred_element_type=f32)
    m1 = m1 + dist * w1c_ref[...] + b1_ref[...]
    h1 = m1 * jax.nn.sigmoid(m1)
    m2 = jnp.dot(h1, w2_ref[...], preferred_element_type=f32) + b2_ref[...]
    msg = m2 * jax.nn.sigmoid(m2)
    gate = msg[:, :H]
    out_ref[0] = msg[:, H:]
    out_ref[1] = gate * d[:, 0:1]
    out_ref[2] = gate * d[:, 1:2]
    out_ref[3] = gate * d[:, 2:3]


def _tc_edge(xr, xc, geom, w1a, w1b, w1c, b1, w2, b2):
    grid = (E_PAD // BE,)
    full = lambda shape: pl.BlockSpec(shape, lambda i: tuple(0 for _ in shape))
    return pl.pallas_call(
        _edge_body,
        grid=grid,
        in_specs=[
            pl.BlockSpec((BE, H), lambda i: (i, 0)),
            pl.BlockSpec((BE, H), lambda i: (i, 0)),
            pl.BlockSpec((BE, 16), lambda i: (i, 0)),
            full((H, 2 * H)),
            full((H, 2 * H)),
            full((1, 2 * H)),
            full((1, 2 * H)),
            full((2 * H, 2 * H)),
            full((1, 2 * H)),
        ],
        out_specs=pl.BlockSpec((4, BE, H), lambda i: (0, i, 0)),
        out_shape=jax.ShapeDtypeStruct((4, E_PAD, H), f32),
    )(xr, xc, geom, w1a, w1b, w1c, b1, w2, b2)


def _node_body(x_ref, agg_ref, v_ref, u1a_ref, u1b_ref, b1_ref, u2_ref,
               b2_ref, vmix_ref, xn_ref, vn_ref):
    x = x_ref[...]
    aggs = agg_ref[0]
    h = (jnp.dot(x, u1a_ref[...], preferred_element_type=f32)
         + jnp.dot(aggs, u1b_ref[...], preferred_element_type=f32)
         + b1_ref[...])
    h = h * jax.nn.sigmoid(h)
    xn_ref[...] = x + jnp.dot(h, u2_ref[...], preferred_element_type=f32) + b2_ref[...]
    for c in range(3):
        vc = v_ref[:, c * H:(c + 1) * H]
        vn_ref[:, c * H:(c + 1) * H] = (
            vc + agg_ref[c + 1]
            + jnp.dot(vc, vmix_ref[...], preferred_element_type=f32))


def _tc_node(x, agg3, v, u1a, u1b, b1, u2, b2, vmix):
    grid = (N_PAD // BN,)
    full = lambda shape: pl.BlockSpec(shape, lambda i: tuple(0 for _ in shape))
    return pl.pallas_call(
        _node_body,
        grid=grid,
        in_specs=[
            pl.BlockSpec((BN, H), lambda i: (i, 0)),
            pl.BlockSpec((4, BN, H), lambda i: (0, i, 0)),
            pl.BlockSpec((BN, 3 * H), lambda i: (i, 0)),
            full((H, H)),
            full((H, H)),
            full((1, H)),
            full((H, H)),
            full((1, H)),
            full((H, H)),
        ],
        out_specs=[
            pl.BlockSpec((BN, H), lambda i: (i, 0)),
            pl.BlockSpec((BN, 3 * H), lambda i: (i, 0)),
        ],
        out_shape=[
            jax.ShapeDtypeStruct((N_PAD, H), f32),
            jax.ShapeDtypeStruct((N_PAD, 3 * H), f32),
        ],
    )(x, agg3, v, u1a, u1b, b1, u2, b2, vmix)


def _pool_body(batch_ref, v_ref, fw_ref, e3_ref, out_ref):
    i = pl.program_id(0)

    @pl.when(i == 0)
    def _():
        out_ref[...] = jnp.zeros_like(out_ref)

    b = batch_ref[0]
    mids = lax.broadcasted_iota(i32, (NUM_MOLS, BN), 0)
    mask = (mids == b).astype(f32)
    fw = fw_ref[...]
    ndp = jnp.zeros((BN, H), f32)
    for c in range(3):
        nd = jnp.sum(v_ref[:, c * H:(c + 1) * H] * fw, axis=1, keepdims=True)
        ndp = ndp + nd * e3_ref[c:c + 1, :]
    out_ref[...] += jnp.dot(mask, ndp, preferred_element_type=f32)


def _tc_pool(batch3, v, fw_row, e3):
    grid = (N_PAD // BN,)
    return pl.pallas_call(
        _pool_body,
        grid=grid,
        in_specs=[
            pl.BlockSpec((1, 1, BN), lambda i: (i, 0, 0)),
            pl.BlockSpec((BN, 3 * H), lambda i: (i, 0)),
            pl.BlockSpec((1, H), lambda i: (0, 0)),
            pl.BlockSpec((8, H), lambda i: (0, 0)),
        ],
        out_specs=pl.BlockSpec((NUM_MOLS, H), lambda i: (0, 0)),
        out_shape=jax.ShapeDtypeStruct((NUM_MOLS, H), f32),
    )(batch3, v, fw_row, e3)


def kernel(z, pos, edge_index, batch, emb_table, m1_w, m1_b, m2_w, m2_b,
           u1_w, u1_b, u2_w, u2_b, vmix_w, final_w):
    n_layers = m1_w.shape[0]

    row = edge_index[0].astype(i32)
    col = edge_index[1].astype(i32)
    epad = E_PAD - N_EDGES
    row_p = jnp.concatenate([row, jnp.full((epad,), N_NODES, i32)])
    col_p = jnp.concatenate([col, jnp.full((epad,), N_NODES, i32)])
    row2 = row_p.reshape(E_PAD // EC, EC)
    col2 = col_p.reshape(E_PAD // EC, EC)

    npad = N_PAD - N_NODES
    z2 = jnp.concatenate([z.astype(i32), jnp.zeros((npad,), i32)]).reshape(-1, NZ_W)
    pos_pad = jnp.zeros((N_PAD, H), f32).at[:N_NODES, :3].set(pos.astype(f32))
    batch_p = jnp.concatenate([batch.astype(i32), jnp.full((npad,), NUM_MOLS, i32)])
    batch3 = batch_p.reshape(N_PAD // BN, 1, BN)
    zeros_n = jnp.zeros((N_PAD, H), f32)
    fw_row = final_w.astype(f32).reshape(1, H)
    e3 = jnp.eye(8, H, dtype=f32)

    x, posr, posc = _sc_gather_init(emb_table.astype(f32), z2, pos_pad,
                                    row2, col2)
    geom = _tc_geom(posr, posc)

    v = jnp.broadcast_to(pos_pad[:, :3, None], (N_PAD, 3, H)).reshape(N_PAD, 3 * H)

    for l in range(n_layers):
        w1a = m1_w[l, :H, :]
        w1b = m1_w[l, H:2 * H, :]
        w1c = m1_w[l, 2 * H:2 * H + 1, :]
        b1 = m1_b[l].reshape(1, 2 * H)
        w2 = m2_w[l]
        b2 = m2_b[l].reshape(1, 2 * H)
        u1a = u1_w[l, :H, :]
        u1b = u1_w[l, H:, :]
        ub1 = u1_b[l].reshape(1, H)
        u2 = u2_w[l]
        ub2 = u2_b[l].reshape(1, H)
        vmix = vmix_w[l]

        xr, xc = _sc_gather_x(x, row2, col2)
        msgs3 = _tc_edge(xr, xc, geom, w1a, w1b, w1c, b1, w2, b2)
        agg3 = _sc_scatter(msgs3, row2, zeros_n)
        x, v = _tc_node(x, agg3, v, u1a, u1b, ub1, u2, ub2, vmix)

    out = _tc_pool(batch3, v, fw_row, e3)
    return out[:NUM_MOLS, :3]
